# baseline (speedup 1.0000x reference)
import functools

import jax
import jax.numpy as jnp
from jax import lax
from jax.experimental import pallas as pl
from jax.experimental.pallas import tpu as pltpu
from jax.experimental.pallas import tpu_sc as plsc

_B, _H, _D = 16384, 50, 64
_V = 1000000
_NC, _NS = 2, 16
_NW = _NC * _NS
_CBW = _B // 128 // _NW
_UNITS = _CBW * _H
_CB = 16384


def _tc_transpose_body(x_ref, o_ref):
    xt = x_ref[...].T.reshape(_CB // 2, 2, 64)
    o_ref[...] = jnp.concatenate([xt[:, 0, :], xt[:, 1, :]], axis=1)


def _linearize_table(table):
    table_t = table.T
    grid = (_V + _CB - 1) // _CB
    lin = pl.pallas_call(
        _tc_transpose_body,
        grid=(grid,),
        in_specs=[pl.BlockSpec((64, _CB), lambda i: (0, i))],
        out_specs=pl.BlockSpec((_CB // 2, 128), lambda i: (i, 0)),
        out_shape=jax.ShapeDtypeStruct((_V // 2, 128), jnp.float32),
    )(table_t)
    return lin.reshape(_V, _D)


def _sc_body(idx_hbm, table_hbm, out_hbm, idx_v, rows_v,
             gsem0, gsem1, osem0, osem1):
    gsems = (gsem0, gsem1)
    osems = (osem0, osem1)
    wid = lax.axis_index("s") * _NC + lax.axis_index("c")

    pltpu.sync_copy(idx_hbm.at[:, pl.ds(wid * _CBW, _CBW), :], idx_v)

    def unit_hc(u):
        return u // _H, lax.rem(u, _H)

    def issue_gather(u, b):
        cc, h = unit_hc(u)
        pltpu.async_copy(table_hbm.at[idx_v.at[h, cc]], rows_v.at[b], gsems[b])

    def wait_gather(b):
        pltpu.make_async_copy(
            table_hbm.at[pl.ds(0, 128)], rows_v.at[b], gsems[b]
        ).wait()

    def issue_write(u, b):
        cc, h = unit_hc(u)
        pltpu.async_copy(
            rows_v.at[b], out_hbm.at[h, wid * _CBW + cc], osems[b])

    def wait_write(b):
        pltpu.make_async_copy(
            out_hbm.at[0, 0], rows_v.at[b], osems[b]).wait()

    issue_gather(0, 0)

    def outer(gi, carry):
        for s in range(2):
            u = 2 * gi + s
            wait_gather(s)

            @pl.when(u >= 1)
            def _():
                wait_write(1 - s)

            @pl.when(u < _UNITS - 1)
            def _():
                issue_gather(u + 1, 1 - s)

            issue_write(u, s)
        return carry

    lax.fori_loop(0, _UNITS // 2, outer, 0, unroll=False)
    wait_write(1)


_G = 16


def _tc_format_body(x_ref, o_ref):
    x = x_ref[...].reshape(_G, 64, 128)
    for cbl in range(_G):
        wa = x[cbl, :, 0:64].T
        wb = x[cbl, :, 64:128].T
        w = jnp.concatenate([wa, wb], axis=1)
        o_ref[0, :, cbl, :, :] = w.reshape(8, 8, 128)


def _format_output(gat):
    x2 = gat.reshape(_H * 8192, 128)
    out5 = pl.pallas_call(
        _tc_format_body,
        grid=(_H, 128 // _G),
        in_specs=[pl.BlockSpec((_G * 64, 128),
                               lambda h, g: (h * (128 // _G) + g, 0))],
        out_specs=pl.BlockSpec((1, 8, _G, 8, 128),
                               lambda h, g: (h, 0, g, 0, 0)),
        out_shape=jax.ShapeDtypeStruct((_H, 8, 128, 8, 128), jnp.float32),
    )(x2)
    return out5


def kernel(inp, table):
    table_lin = _linearize_table(table)
    idx3 = inp.astype(jnp.int32).T.reshape(_H, _B // 128, 128)
    idx3 = jnp.stack([idx3[..., :64], idx3[..., 64:]], axis=-1)
    idx3 = idx3.reshape(_H, _B // 128, 128)
    mesh = plsc.VectorSubcoreMesh(core_axis_name="c", subcore_axis_name="s")
    gat = pl.kernel(
        _sc_body,
        out_type=jax.ShapeDtypeStruct((_H, _B // 128, 128, _D), jnp.float32),
        mesh=mesh,
        compiler_params=pltpu.CompilerParams(
            use_tc_tiling_on_sc=False, needs_layout_passes=False),
        scratch_types=[
            pltpu.VMEM((_H, _CBW, 128), jnp.int32),
            pltpu.VMEM((2, 128, _D), jnp.float32),
            pltpu.SemaphoreType.DMA,
            pltpu.SemaphoreType.DMA,
            pltpu.SemaphoreType.DMA,
            pltpu.SemaphoreType.DMA,
        ],
    )(idx3, table_lin)
    out5 = _format_output(gat)
    flat = out5.reshape(_B * _H * _D)
    res = flat.reshape(_H, 8, 128, 8, 128).transpose(2, 4, 0, 1, 3)
    return res.reshape(_B, _H, _D)


# speedup vs baseline: 1.4323x; 1.4323x over previous
import functools

import jax
import jax.numpy as jnp
from jax import lax
from jax.experimental import pallas as pl
from jax.experimental.pallas import tpu as pltpu
from jax.experimental.pallas import tpu_sc as plsc

_B, _H, _D = 16384, 50, 64
_V = 1000000
_NC, _NS = 2, 16
_NW = _NC * _NS
_CBW = _B // 128 // _NW
_UNITS = _CBW * _H
_CB = 16384


def _tc_transpose_body(x_ref, o_ref):
    xt = x_ref[...].T.reshape(_CB // 2, 2, 64)
    o_ref[...] = jnp.concatenate([xt[:, 0, :], xt[:, 1, :]], axis=1)


def _linearize_table(table):
    table_t = table.T
    grid = (_V + _CB - 1) // _CB
    lin = pl.pallas_call(
        _tc_transpose_body,
        grid=(grid,),
        in_specs=[pl.BlockSpec((64, _CB), lambda i: (0, i))],
        out_specs=pl.BlockSpec((_CB // 2, 128), lambda i: (i, 0)),
        out_shape=jax.ShapeDtypeStruct((_V // 2, 128), jnp.float32),
    )(table_t)
    return lin.reshape(_V, _D)


def _sc_body(idx_hbm, table_hbm, out_hbm, idx_v, rows_v,
             gsem0, gsem1, osem0, osem1):
    gsems = (gsem0, gsem1)
    osems = (osem0, osem1)
    wid = lax.axis_index("s") * _NC + lax.axis_index("c")

    pltpu.sync_copy(idx_hbm.at[:, pl.ds(wid * _CBW, _CBW), :], idx_v)

    def unit_hc(u):
        return u // _H, lax.rem(u, _H)

    def issue_gather(u, b):
        cc, h = unit_hc(u)
        pltpu.async_copy(table_hbm.at[idx_v.at[h, cc]], rows_v.at[b], gsems[b])

    def wait_gather(b):
        pltpu.make_async_copy(
            table_hbm.at[pl.ds(0, 128)], rows_v.at[b], gsems[b]
        ).wait()

    def issue_write(u, b):
        cc, h = unit_hc(u)
        pltpu.async_copy(
            rows_v.at[b], out_hbm.at[h, wid * _CBW + cc], osems[b])

    def wait_write(b):
        pltpu.make_async_copy(
            out_hbm.at[0, 0], rows_v.at[b], osems[b]).wait()

    issue_gather(0, 0)

    def outer(gi, carry):
        for s in range(2):
            u = 2 * gi + s
            wait_gather(s)

            @pl.when(u >= 1)
            def _():
                wait_write(1 - s)

            @pl.when(u < _UNITS - 1)
            def _():
                issue_gather(u + 1, 1 - s)

            issue_write(u, s)
        return carry

    lax.fori_loop(0, _UNITS // 2, outer, 0, unroll=False)
    wait_write(1)


_G = 16


def _tc_format_body(x_ref, o_ref):
    xt = x_ref[...].T
    lo = xt[0:64].reshape(64, _G, 64)
    hi = xt[64:128].reshape(64, _G, 64)
    w = jnp.concatenate([lo, hi], axis=2)
    y = w.reshape(8, 8, _G, 128).transpose(0, 2, 1, 3)
    o_ref[...] = y[None]


def _format_output(gat):
    x2 = gat.reshape(_H * 8192, 128)
    out5 = pl.pallas_call(
        _tc_format_body,
        grid=(_H, 128 // _G),
        in_specs=[pl.BlockSpec((_G * 64, 128),
                               lambda h, g: (h * (128 // _G) + g, 0))],
        out_specs=pl.BlockSpec((1, 8, _G, 8, 128),
                               lambda h, g: (h, 0, g, 0, 0)),
        out_shape=jax.ShapeDtypeStruct((_H, 8, 128, 8, 128), jnp.float32),
    )(x2)
    return out5


def kernel(inp, table):
    table_lin = _linearize_table(table)
    idx3 = inp.astype(jnp.int32).T.reshape(_H, _B // 128, 128)
    idx3 = jnp.stack([idx3[..., :64], idx3[..., 64:]], axis=-1)
    idx3 = idx3.reshape(_H, _B // 128, 128)
    mesh = plsc.VectorSubcoreMesh(core_axis_name="c", subcore_axis_name="s")
    gat = pl.kernel(
        _sc_body,
        out_type=jax.ShapeDtypeStruct((_H, _B // 128, 128, _D), jnp.float32),
        mesh=mesh,
        compiler_params=pltpu.CompilerParams(
            use_tc_tiling_on_sc=False, needs_layout_passes=False),
        scratch_types=[
            pltpu.VMEM((_H, _CBW, 128), jnp.int32),
            pltpu.VMEM((2, 128, _D), jnp.float32),
            pltpu.SemaphoreType.DMA,
            pltpu.SemaphoreType.DMA,
            pltpu.SemaphoreType.DMA,
            pltpu.SemaphoreType.DMA,
        ],
    )(idx3, table_lin)
    out5 = _format_output(gat)
    flat = out5.reshape(_B * _H * _D)
    res = flat.reshape(_H, 8, 128, 8, 128).transpose(2, 4, 0, 1, 3)
    return res.reshape(_B, _H, _D)


# speedup vs baseline: 1.5800x; 1.1031x over previous
import functools

import jax
import jax.numpy as jnp
from jax import lax
from jax.experimental import pallas as pl
from jax.experimental.pallas import tpu as pltpu
from jax.experimental.pallas import tpu_sc as plsc

_B, _H, _D = 16384, 50, 64
_V = 1000000
_NC, _NS = 2, 16
_NW = _NC * _NS
_CBW = _B // 128 // _NW
_UNITS = _CBW * _H
_CB = 16384


def _tc_transpose_body(x_ref, o_ref):
    xt = x_ref[...].T.reshape(_CB // 2, 2, 64)
    o_ref[...] = jnp.concatenate([xt[:, 0, :], xt[:, 1, :]], axis=1)


def _linearize_table(table):
    table_t = table.T
    grid = (_V + _CB - 1) // _CB
    lin = pl.pallas_call(
        _tc_transpose_body,
        grid=(grid,),
        in_specs=[pl.BlockSpec((64, _CB), lambda i: (0, i))],
        out_specs=pl.BlockSpec((_CB // 2, 128), lambda i: (i, 0)),
        out_shape=jax.ShapeDtypeStruct((_V // 2, 128), jnp.float32),
    )(table_t)
    return lin.reshape(_V, _D)


def _sc_body(idx_hbm, table_hbm, out_hbm, idx_v, rows_v,
             gsem0, gsem1, osem0, osem1):
    gsems = (gsem0, gsem1)
    osems = (osem0, osem1)
    wid = lax.axis_index("s") * _NC + lax.axis_index("c")

    pltpu.sync_copy(idx_hbm.at[:, pl.ds(wid * _CBW, _CBW), :], idx_v)

    def unit_hc(u):
        return u // _H, lax.rem(u, _H)

    def issue_gather(u, b):
        cc, h = unit_hc(u)
        pltpu.async_copy(table_hbm.at[idx_v.at[h, cc]], rows_v.at[b], gsems[b])

    def wait_gather(b):
        pltpu.make_async_copy(
            table_hbm.at[pl.ds(0, 128)], rows_v.at[b], gsems[b]
        ).wait()

    def issue_write(u, b):
        cc, h = unit_hc(u)
        pltpu.async_copy(
            rows_v.at[b], out_hbm.at[h, wid * _CBW + cc], osems[b])

    def wait_write(b):
        pltpu.make_async_copy(
            out_hbm.at[0, 0], rows_v.at[b], osems[b]).wait()

    issue_gather(0, 0)

    def outer(gi, carry):
        for s in range(2):
            u = 2 * gi + s
            wait_gather(s)

            @pl.when(u >= 1)
            def _():
                wait_write(1 - s)

            @pl.when(u < _UNITS - 1)
            def _():
                issue_gather(u + 1, 1 - s)

            issue_write(u, s)
        return carry

    lax.fori_loop(0, _UNITS // 2, outer, 0, unroll=False)
    wait_write(1)


_G = 32


def _tc_format_body(x_ref, o_ref):
    xt = x_ref[...].T
    lo = xt[0:64].reshape(64, _G, 64)
    hi = xt[64:128].reshape(64, _G, 64)
    w = jnp.concatenate([lo, hi], axis=2)
    y = w.reshape(8, 8, _G, 128).transpose(0, 2, 1, 3)
    o_ref[...] = y[None]


def _format_output(gat):
    x2 = gat.reshape(_H * 8192, 128)
    out5 = pl.pallas_call(
        _tc_format_body,
        grid=(_H, 128 // _G),
        in_specs=[pl.BlockSpec((_G * 64, 128),
                               lambda h, g: (h * (128 // _G) + g, 0))],
        out_specs=pl.BlockSpec((1, 8, _G, 8, 128),
                               lambda h, g: (h, 0, g, 0, 0)),
        out_shape=jax.ShapeDtypeStruct((_H, 8, 128, 8, 128), jnp.float32),
    )(x2)
    return out5


def kernel(inp, table):
    table_lin = _linearize_table(table)
    idx3 = inp.astype(jnp.int32).T.reshape(_H, _B // 128, 128)
    idx3 = jnp.stack([idx3[..., :64], idx3[..., 64:]], axis=-1)
    idx3 = idx3.reshape(_H, _B // 128, 128)
    mesh = plsc.VectorSubcoreMesh(core_axis_name="c", subcore_axis_name="s")
    gat = pl.kernel(
        _sc_body,
        out_type=jax.ShapeDtypeStruct((_H, _B // 128, 128, _D), jnp.float32),
        mesh=mesh,
        compiler_params=pltpu.CompilerParams(
            use_tc_tiling_on_sc=False, needs_layout_passes=False),
        scratch_types=[
            pltpu.VMEM((_H, _CBW, 128), jnp.int32),
            pltpu.VMEM((2, 128, _D), jnp.float32),
            pltpu.SemaphoreType.DMA,
            pltpu.SemaphoreType.DMA,
            pltpu.SemaphoreType.DMA,
            pltpu.SemaphoreType.DMA,
        ],
    )(idx3, table_lin)
    out5 = _format_output(gat)
    flat = out5.reshape(_B * _H * _D)
    res = flat.reshape(_H, 8, 128, 8, 128).transpose(2, 4, 0, 1, 3)
    return res.reshape(_B, _H, _D)


# speedup vs baseline: 1.6808x; 1.0638x over previous
import functools

import jax
import jax.numpy as jnp
from jax import lax
from jax.experimental import pallas as pl
from jax.experimental.pallas import tpu as pltpu
from jax.experimental.pallas import tpu_sc as plsc

_B, _H, _D = 16384, 50, 64
_V = 1000000
_NC, _NS = 2, 16
_NW = _NC * _NS
_CBW = _B // 128 // _NW
_UNITS = _CBW * _H
_CB = 16384


def _tc_transpose_body(x_ref, o_ref):
    xt = x_ref[...].T.reshape(_CB // 2, 2, 64)
    o_ref[...] = jnp.concatenate([xt[:, 0, :], xt[:, 1, :]], axis=1)


def _linearize_table(table):
    table_t = table.T
    grid = (_V + _CB - 1) // _CB
    lin = pl.pallas_call(
        _tc_transpose_body,
        grid=(grid,),
        in_specs=[pl.BlockSpec((64, _CB), lambda i: (0, i))],
        out_specs=pl.BlockSpec((_CB // 2, 128), lambda i: (i, 0)),
        out_shape=jax.ShapeDtypeStruct((_V // 2, 128), jnp.float32),
    )(table_t)
    return lin.reshape(_V, _D)


def _sc_body(idx_hbm, table_hbm, out_hbm, idx_v, rows_v,
             gsem0, gsem1, osem0, osem1):
    gsems = (gsem0, gsem1)
    osems = (osem0, osem1)
    wid = lax.axis_index("s") * _NC + lax.axis_index("c")

    pltpu.sync_copy(idx_hbm.at[:, pl.ds(wid * _CBW, _CBW), :], idx_v)

    def unit_hc(u):
        return u // _H, lax.rem(u, _H)

    def issue_gather(u, b):
        cc, h = unit_hc(u)
        pltpu.async_copy(table_hbm.at[idx_v.at[h, cc]], rows_v.at[b], gsems[b])

    def wait_gather(b):
        pltpu.make_async_copy(
            table_hbm.at[pl.ds(0, 128)], rows_v.at[b], gsems[b]
        ).wait()

    def issue_write(u, b):
        cc, h = unit_hc(u)
        pltpu.async_copy(
            rows_v.at[b], out_hbm.at[h, wid * _CBW + cc], osems[b])

    def wait_write(b):
        pltpu.make_async_copy(
            out_hbm.at[0, 0], rows_v.at[b], osems[b]).wait()

    issue_gather(0, 0)

    def outer(gi, carry):
        for s in range(2):
            u = 2 * gi + s
            wait_gather(s)

            @pl.when(u >= 1)
            def _():
                wait_write(1 - s)

            @pl.when(u < _UNITS - 1)
            def _():
                issue_gather(u + 1, 1 - s)

            issue_write(u, s)
        return carry

    lax.fori_loop(0, _UNITS // 2, outer, 0, unroll=False)
    wait_write(1)


_G = 64


def _tc_format_body(x_ref, o_ref):
    xt = x_ref[...].T
    lo = xt[0:64].reshape(64, _G, 64)
    hi = xt[64:128].reshape(64, _G, 64)
    w = jnp.concatenate([lo, hi], axis=2)
    y = w.reshape(8, 8, _G, 128).transpose(0, 2, 1, 3)
    o_ref[...] = y[None]


def _format_output(gat):
    x2 = gat.reshape(_H * 8192, 128)
    out5 = pl.pallas_call(
        _tc_format_body,
        grid=(_H, 128 // _G),
        in_specs=[pl.BlockSpec((_G * 64, 128),
                               lambda h, g: (h * (128 // _G) + g, 0))],
        out_specs=pl.BlockSpec((1, 8, _G, 8, 128),
                               lambda h, g: (h, 0, g, 0, 0)),
        out_shape=jax.ShapeDtypeStruct((_H, 8, 128, 8, 128), jnp.float32),
    )(x2)
    return out5


def kernel(inp, table):
    table_lin = _linearize_table(table)
    idx3 = inp.astype(jnp.int32).T.reshape(_H, _B // 128, 128)
    idx3 = jnp.stack([idx3[..., :64], idx3[..., 64:]], axis=-1)
    idx3 = idx3.reshape(_H, _B // 128, 128)
    mesh = plsc.VectorSubcoreMesh(core_axis_name="c", subcore_axis_name="s")
    gat = pl.kernel(
        _sc_body,
        out_type=jax.ShapeDtypeStruct((_H, _B // 128, 128, _D), jnp.float32),
        mesh=mesh,
        compiler_params=pltpu.CompilerParams(
            use_tc_tiling_on_sc=False, needs_layout_passes=False),
        scratch_types=[
            pltpu.VMEM((_H, _CBW, 128), jnp.int32),
            pltpu.VMEM((2, 128, _D), jnp.float32),
            pltpu.SemaphoreType.DMA,
            pltpu.SemaphoreType.DMA,
            pltpu.SemaphoreType.DMA,
            pltpu.SemaphoreType.DMA,
        ],
    )(idx3, table_lin)
    out5 = _format_output(gat)
    flat = out5.reshape(_B * _H * _D)
    res = flat.reshape(_H, 8, 128, 8, 128).transpose(2, 4, 0, 1, 3)
    return res.reshape(_B, _H, _D)
